# baseline (speedup 1.0000x reference)
import jax
import jax.numpy as jnp
from jax.experimental import pallas as pl
from jax.experimental.pallas import tpu as pltpu

_EPS = 1e-5
_T = 2048


def _first(b, n):
    return (b == 0) & (n == 0)


def _mm(a, b):
    return jax.lax.dot_general(
        a.astype(jnp.bfloat16), b.astype(jnp.bfloat16),
        (((1,), (0,)), ((), ())), preferred_element_type=jnp.float32)


def _outer(a):
    ab = a.astype(jnp.bfloat16)
    return jax.lax.dot_general(ab, ab, (((1,), (1,)), ((), ())),
                               preferred_element_type=jnp.float32)


def _p0_kernel(x_ref, mom_ref, sum_ref):
    x = x_ref[...]
    m = jax.lax.dot_general(x, x, (((1,), (1,)), ((), ())),
                            preferred_element_type=jnp.float32)
    s = jnp.sum(x, axis=1, keepdims=True)
    f = _first(pl.program_id(0), pl.program_id(1))

    @pl.when(f)
    def _():
        mom_ref[...] = m
        sum_ref[...] = s

    @pl.when(jnp.logical_not(f))
    def _():
        mom_ref[...] = mom_ref[...] + m
        sum_ref[...] = sum_ref[...] + s


def _u_of(x_ref, au_ref, cu_ref):
    return jnp.maximum(_mm(au_ref[...], x_ref[...]) + cu_ref[...], 0.0)


def _p1_kernel(p_ref, x_ref, au_ref, cu_ref, out_ref):
    u = _u_of(x_ref, au_ref, cu_ref)
    p = p_ref[...]
    ones = jnp.ones((1, p.shape[1]), jnp.float32)
    ya = jnp.concatenate([p, u, ones], axis=0)
    m = _outer(ya)
    f = _first(pl.program_id(0), pl.program_id(1))

    @pl.when(f)
    def _():
        out_ref[...] = m

    @pl.when(jnp.logical_not(f))
    def _():
        out_ref[...] = out_ref[...] + m


def _x3_of(p_ref, x_ref, au_ref, cu_ref, a3p_ref, a3u_ref, a3c_ref):
    u = _u_of(x_ref, au_ref, cu_ref)
    h = _mm(a3p_ref[...], p_ref[...]) + _mm(a3u_ref[...], u) + a3c_ref[...]
    return jnp.maximum(h, 0.0)


def _p2_kernel(p_ref, x_ref, au_ref, cu_ref, a3p_ref, a3u_ref, a3c_ref,
               out_ref):
    x3 = _x3_of(p_ref, x_ref, au_ref, cu_ref, a3p_ref, a3u_ref, a3c_ref)
    ones = jnp.ones((1, x3.shape[1]), jnp.float32)
    xa = jnp.concatenate([x3, ones], axis=0)
    m = _outer(xa)
    f = _first(pl.program_id(0), pl.program_id(1))

    @pl.when(f)
    def _():
        out_ref[...] = m

    @pl.when(jnp.logical_not(f))
    def _():
        out_ref[...] = out_ref[...] + m


def _p3_kernel(p_ref, x_ref, au_ref, cu_ref, a3p_ref, a3u_ref, a3c_ref,
               a4_ref, a4c_ref, out_ref):
    x3 = _x3_of(p_ref, x_ref, au_ref, cu_ref, a3p_ref, a3u_ref, a3c_ref)
    x4 = jnp.maximum(_mm(a4_ref[...], x3) + a4c_ref[...], 0.0)
    ones = jnp.ones((1, x4.shape[1]), jnp.float32)
    xa = jnp.concatenate([x4, ones], axis=0)
    m = _outer(xa)
    f = pl.program_id(1) == 0

    @pl.when(f)
    def _():
        out_ref[...] = m

    @pl.when(jnp.logical_not(f))
    def _():
        out_ref[...] = out_ref[...] + m


def _p4_kernel(p_ref, x_ref, au_ref, cu_ref, a3p_ref, a3u_ref, a3c_ref,
               a4_ref, a4c_ref, wf_ref, out_ref):
    x3 = _x3_of(p_ref, x_ref, au_ref, cu_ref, a3p_ref, a3u_ref, a3c_ref)
    x4 = jnp.maximum(_mm(a4_ref[...], x3) + a4c_ref[...], 0.0)
    z5 = _mm(wf_ref[...], x4)
    pm = jnp.max(z5, axis=1, keepdims=True)
    f = pl.program_id(1) == 0

    @pl.when(f)
    def _():
        out_ref[...] = pm

    @pl.when(jnp.logical_not(f))
    def _():
        out_ref[...] = jnp.maximum(out_ref[...], pm)


def _qdiag(A, M):
    return jnp.sum((A @ M) * A, axis=1)


def kernel(xyz, points, W0, b0, g0, be0, W1, b1, g1, be1, W2, b2, g2, be2,
           Wc0, bc0, gc0, bec0, Wc1, bc1, gc1, bec1, wk, We, bE, gE, beE):
    B, _, N = xyz.shape
    Cin = points.shape[1]
    T = _T
    NB = N // T
    cnt = float(B * N)
    cp = pltpu.CompilerParams(dimension_semantics=("arbitrary", "arbitrary"))

    xyz = xyz.astype(jnp.float32)
    points = points.astype(jnp.float32)

    mom_x, sum_x = pl.pallas_call(
        _p0_kernel,
        grid=(B, NB),
        in_specs=[pl.BlockSpec((None, 3, T), lambda b, n: (b, 0, n))],
        out_specs=[pl.BlockSpec((3, 3), lambda b, n: (0, 0)),
                   pl.BlockSpec((3, 1), lambda b, n: (0, 0))],
        out_shape=[jax.ShapeDtypeStruct((3, 3), jnp.float32),
                   jax.ShapeDtypeStruct((3, 1), jnp.float32)],
        compiler_params=cp,
    )(xyz)

    mean_x = sum_x[:, 0] / cnt
    Mx = mom_x / cnt
    m0 = W0 @ mean_x + b0
    Ez0 = _qdiag(W0, Mx) + 2.0 * b0 * (W0 @ mean_x) + b0 * b0
    s0 = jnp.sqrt(Ez0 - m0 * m0 + _EPS)
    Au = (g0 / s0)[:, None] * W0
    cu = (g0 * (b0 - m0) / s0 + be0)[:, None]

    maug = pl.pallas_call(
        _p1_kernel,
        grid=(B, NB),
        in_specs=[
            pl.BlockSpec((None, Cin, T), lambda b, n: (b, 0, n)),
            pl.BlockSpec((None, 3, T), lambda b, n: (b, 0, n)),
            pl.BlockSpec((8, 3), lambda b, n: (0, 0)),
            pl.BlockSpec((8, 1), lambda b, n: (0, 0)),
        ],
        out_specs=pl.BlockSpec((73, 73), lambda b, n: (0, 0)),
        out_shape=jax.ShapeDtypeStruct((73, 73), jnp.float32),
        compiler_params=cp,
    )(points, xyz, Au, cu)

    MY = maug[:72, :72] / cnt
    meanY = maug[72, :72] / cnt

    mean_u = meanY[64:]
    Muu = MY[64:, 64:]
    m1 = W1 @ mean_u + b1
    Ez1 = _qdiag(W1, Muu) + 2.0 * b1 * (W1 @ mean_u) + b1 * b1
    s1 = jnp.sqrt(Ez1 - m1 * m1 + _EPS)
    W1f = (g1 / s1)[:, None] * W1
    c1f = g1 * (b1 - m1) / s1 + be1

    A = jnp.concatenate([W2, W2 @ W1f], axis=1)
    a = W2 @ c1f + b2
    m2 = A @ meanY + a
    Ez2 = _qdiag(A, MY) + 2.0 * a * (A @ meanY) + a * a
    s2 = jnp.sqrt(Ez2 - m2 * m2 + _EPS)
    A2 = (g2 / s2)[:, None] * A
    a2 = g2 * (a - m2) / s2 + be2

    A3 = Wc0 @ A2
    a3 = Wc0 @ a2 + bc0
    m3 = A3 @ meanY + a3
    Ez3 = _qdiag(A3, MY) + 2.0 * a3 * (A3 @ meanY) + a3 * a3
    s3 = jnp.sqrt(Ez3 - m3 * m3 + _EPS)
    A3f = (gc0 / s3)[:, None] * A3
    a3f = (gc0 * (a3 - m3) / s3 + bec0)[:, None]
    A3p = A3f[:, :64]
    A3u = A3f[:, 64:]

    m3aug = pl.pallas_call(
        _p2_kernel,
        grid=(B, NB),
        in_specs=[
            pl.BlockSpec((None, Cin, T), lambda b, n: (b, 0, n)),
            pl.BlockSpec((None, 3, T), lambda b, n: (b, 0, n)),
            pl.BlockSpec((8, 3), lambda b, n: (0, 0)),
            pl.BlockSpec((8, 1), lambda b, n: (0, 0)),
            pl.BlockSpec((64, 64), lambda b, n: (0, 0)),
            pl.BlockSpec((64, 8), lambda b, n: (0, 0)),
            pl.BlockSpec((64, 1), lambda b, n: (0, 0)),
        ],
        out_specs=pl.BlockSpec((65, 65), lambda b, n: (0, 0)),
        out_shape=jax.ShapeDtypeStruct((65, 65), jnp.float32),
        compiler_params=cp,
    )(points, xyz, Au, cu, A3p, A3u, a3f)

    mean3 = m3aug[64, :64] / cnt
    M3 = m3aug[:64, :64] / cnt
    m4 = Wc1 @ mean3 + bc1
    Ez4 = _qdiag(Wc1, M3) + 2.0 * bc1 * (Wc1 @ mean3) + bc1 * bc1
    s4 = jnp.sqrt(Ez4 - m4 * m4 + _EPS)
    A4 = (gc1 / s4)[:, None] * Wc1
    a4f = (gc1 * (bc1 - m4) / s4 + bec1)[:, None]

    m4aug = pl.pallas_call(
        _p3_kernel,
        grid=(B, NB),
        in_specs=[
            pl.BlockSpec((None, Cin, T), lambda b, n: (b, 0, n)),
            pl.BlockSpec((None, 3, T), lambda b, n: (b, 0, n)),
            pl.BlockSpec((8, 3), lambda b, n: (0, 0)),
            pl.BlockSpec((8, 1), lambda b, n: (0, 0)),
            pl.BlockSpec((64, 64), lambda b, n: (0, 0)),
            pl.BlockSpec((64, 8), lambda b, n: (0, 0)),
            pl.BlockSpec((64, 1), lambda b, n: (0, 0)),
            pl.BlockSpec((64, 64), lambda b, n: (0, 0)),
            pl.BlockSpec((64, 1), lambda b, n: (0, 0)),
        ],
        out_specs=pl.BlockSpec((None, 65, 65), lambda b, n: (b, 0, 0)),
        out_shape=jax.ShapeDtypeStruct((B, 65, 65), jnp.float32),
        compiler_params=cp,
    )(points, xyz, Au, cu, A3p, A3u, a3f, A4, a4f)

    y_b = m4aug[:, 64, :64] / float(N)
    M4 = m4aug[:, :64, :64] / float(N)

    yp = jnp.pad(y_b, ((0, 0), (1, 1)))
    yc = wk[0] * yp[:, :-2] + wk[1] * yp[:, 1:-1] + wk[2] * yp[:, 2:]
    sig = jax.nn.sigmoid(yc)

    Web = We[None, :, :] * sig[:, None, :]
    mE_b = jnp.einsum('boc,bc->bo', Web, y_b) + bE[None, :]
    mE = jnp.mean(mE_b, axis=0)
    Ez5 = jnp.mean(
        jnp.einsum('boc,bcd,bod->bo', Web, M4, Web)
        + 2.0 * bE[None, :] * (mE_b - bE[None, :]) + (bE * bE)[None, :],
        axis=0)
    sE = jnp.sqrt(Ez5 - mE * mE + _EPS)
    Wfb = (gE / sE)[None, :, None] * Web
    shift = gE * (bE - mE) / sE + beE

    rawmax = pl.pallas_call(
        _p4_kernel,
        grid=(B, NB),
        in_specs=[
            pl.BlockSpec((None, Cin, T), lambda b, n: (b, 0, n)),
            pl.BlockSpec((None, 3, T), lambda b, n: (b, 0, n)),
            pl.BlockSpec((8, 3), lambda b, n: (0, 0)),
            pl.BlockSpec((8, 1), lambda b, n: (0, 0)),
            pl.BlockSpec((64, 64), lambda b, n: (0, 0)),
            pl.BlockSpec((64, 8), lambda b, n: (0, 0)),
            pl.BlockSpec((64, 1), lambda b, n: (0, 0)),
            pl.BlockSpec((64, 64), lambda b, n: (0, 0)),
            pl.BlockSpec((64, 1), lambda b, n: (0, 0)),
            pl.BlockSpec((None, 128, 64), lambda b, n: (b, 0, 0)),
        ],
        out_specs=pl.BlockSpec((None, 128, 1), lambda b, n: (b, 0, 0)),
        out_shape=jax.ShapeDtypeStruct((B, 128, 1), jnp.float32),
        compiler_params=cp,
    )(points, xyz, Au, cu, A3p, A3u, a3f, A4, a4f, Wfb)

    new_features = rawmax + shift[None, :, None]
    new_xyz = jnp.zeros((B, 3, 1), dtype=xyz.dtype)
    return new_xyz, new_features


# speedup vs baseline: 2.3261x; 2.3261x over previous
import jax
import jax.numpy as jnp
from jax.experimental import pallas as pl
from jax.experimental.pallas import tpu as pltpu

_EPS = 1e-5
_T = 8192


def _first(b, n):
    return (b == 0) & (n == 0)


def _mm(a, b):
    return jax.lax.dot_general(
        a.astype(jnp.bfloat16), b.astype(jnp.bfloat16),
        (((1,), (0,)), ((), ())), preferred_element_type=jnp.float32)


def _outer(a):
    ab = a.astype(jnp.bfloat16)
    return jax.lax.dot_general(ab, ab, (((1,), (1,)), ((), ())),
                               preferred_element_type=jnp.float32)


def _p0_kernel(x_ref, mom_ref, sum_ref):
    x = x_ref[...]
    m = jax.lax.dot_general(x, x, (((1,), (1,)), ((), ())),
                            preferred_element_type=jnp.float32)
    s = jnp.sum(x, axis=1, keepdims=True)
    f = _first(pl.program_id(0), pl.program_id(1))

    @pl.when(f)
    def _():
        mom_ref[...] = m
        sum_ref[...] = s

    @pl.when(jnp.logical_not(f))
    def _():
        mom_ref[...] = mom_ref[...] + m
        sum_ref[...] = sum_ref[...] + s


def _u_of(x_ref, au_ref, cu_ref):
    return jnp.maximum(_mm(au_ref[...], x_ref[...]) + cu_ref[...], 0.0)


def _p1_kernel(p_ref, x_ref, au_ref, cu_ref, out_ref):
    u = _u_of(x_ref, au_ref, cu_ref)
    p = p_ref[...]
    ones = jnp.ones((1, p.shape[1]), jnp.float32)
    ya = jnp.concatenate([p, u, ones], axis=0)
    m = _outer(ya)
    f = _first(pl.program_id(0), pl.program_id(1))

    @pl.when(f)
    def _():
        out_ref[...] = m

    @pl.when(jnp.logical_not(f))
    def _():
        out_ref[...] = out_ref[...] + m


def _x3_of(p_ref, x_ref, au_ref, cu_ref, a3p_ref, a3u_ref, a3c_ref):
    u = _u_of(x_ref, au_ref, cu_ref)
    h = _mm(a3p_ref[...], p_ref[...]) + _mm(a3u_ref[...], u) + a3c_ref[...]
    return jnp.maximum(h, 0.0)


def _p2_kernel(p_ref, x_ref, au_ref, cu_ref, a3p_ref, a3u_ref, a3c_ref,
               out_ref):
    x3 = _x3_of(p_ref, x_ref, au_ref, cu_ref, a3p_ref, a3u_ref, a3c_ref)
    ones = jnp.ones((1, x3.shape[1]), jnp.float32)
    xa = jnp.concatenate([x3, ones], axis=0)
    m = _outer(xa)
    f = _first(pl.program_id(0), pl.program_id(1))

    @pl.when(f)
    def _():
        out_ref[...] = m

    @pl.when(jnp.logical_not(f))
    def _():
        out_ref[...] = out_ref[...] + m


def _p3_kernel(p_ref, x_ref, au_ref, cu_ref, a3p_ref, a3u_ref, a3c_ref,
               a4_ref, a4c_ref, out_ref):
    x3 = _x3_of(p_ref, x_ref, au_ref, cu_ref, a3p_ref, a3u_ref, a3c_ref)
    x4 = jnp.maximum(_mm(a4_ref[...], x3) + a4c_ref[...], 0.0)
    ones = jnp.ones((1, x4.shape[1]), jnp.float32)
    xa = jnp.concatenate([x4, ones], axis=0)
    m = _outer(xa)
    f = pl.program_id(1) == 0

    @pl.when(f)
    def _():
        out_ref[...] = m

    @pl.when(jnp.logical_not(f))
    def _():
        out_ref[...] = out_ref[...] + m


def _p4_kernel(p_ref, x_ref, au_ref, cu_ref, a3p_ref, a3u_ref, a3c_ref,
               a4_ref, a4c_ref, wf_ref, out_ref):
    x3 = _x3_of(p_ref, x_ref, au_ref, cu_ref, a3p_ref, a3u_ref, a3c_ref)
    x4 = jnp.maximum(_mm(a4_ref[...], x3) + a4c_ref[...], 0.0)
    z5 = _mm(wf_ref[...], x4)
    pm = jnp.max(z5, axis=1, keepdims=True)
    f = pl.program_id(1) == 0

    @pl.when(f)
    def _():
        out_ref[...] = pm

    @pl.when(jnp.logical_not(f))
    def _():
        out_ref[...] = jnp.maximum(out_ref[...], pm)


def _qdiag(A, M):
    return jnp.sum((A @ M) * A, axis=1)


def kernel(xyz, points, W0, b0, g0, be0, W1, b1, g1, be1, W2, b2, g2, be2,
           Wc0, bc0, gc0, bec0, Wc1, bc1, gc1, bec1, wk, We, bE, gE, beE):
    B, _, N = xyz.shape
    Cin = points.shape[1]
    T = _T
    NB = N // T
    cnt = float(B * N)
    cp = pltpu.CompilerParams(dimension_semantics=("arbitrary", "arbitrary"))

    xyz = xyz.astype(jnp.float32)
    points = points.astype(jnp.float32)

    mom_x, sum_x = pl.pallas_call(
        _p0_kernel,
        grid=(B, NB),
        in_specs=[pl.BlockSpec((None, 3, T), lambda b, n: (b, 0, n))],
        out_specs=[pl.BlockSpec((3, 3), lambda b, n: (0, 0)),
                   pl.BlockSpec((3, 1), lambda b, n: (0, 0))],
        out_shape=[jax.ShapeDtypeStruct((3, 3), jnp.float32),
                   jax.ShapeDtypeStruct((3, 1), jnp.float32)],
        compiler_params=cp,
    )(xyz)

    mean_x = sum_x[:, 0] / cnt
    Mx = mom_x / cnt
    m0 = W0 @ mean_x + b0
    Ez0 = _qdiag(W0, Mx) + 2.0 * b0 * (W0 @ mean_x) + b0 * b0
    s0 = jnp.sqrt(Ez0 - m0 * m0 + _EPS)
    Au = (g0 / s0)[:, None] * W0
    cu = (g0 * (b0 - m0) / s0 + be0)[:, None]

    maug = pl.pallas_call(
        _p1_kernel,
        grid=(B, NB),
        in_specs=[
            pl.BlockSpec((None, Cin, T), lambda b, n: (b, 0, n)),
            pl.BlockSpec((None, 3, T), lambda b, n: (b, 0, n)),
            pl.BlockSpec((8, 3), lambda b, n: (0, 0)),
            pl.BlockSpec((8, 1), lambda b, n: (0, 0)),
        ],
        out_specs=pl.BlockSpec((73, 73), lambda b, n: (0, 0)),
        out_shape=jax.ShapeDtypeStruct((73, 73), jnp.float32),
        compiler_params=cp,
    )(points, xyz, Au, cu)

    MY = maug[:72, :72] / cnt
    meanY = maug[72, :72] / cnt

    mean_u = meanY[64:]
    Muu = MY[64:, 64:]
    m1 = W1 @ mean_u + b1
    Ez1 = _qdiag(W1, Muu) + 2.0 * b1 * (W1 @ mean_u) + b1 * b1
    s1 = jnp.sqrt(Ez1 - m1 * m1 + _EPS)
    W1f = (g1 / s1)[:, None] * W1
    c1f = g1 * (b1 - m1) / s1 + be1

    A = jnp.concatenate([W2, W2 @ W1f], axis=1)
    a = W2 @ c1f + b2
    m2 = A @ meanY + a
    Ez2 = _qdiag(A, MY) + 2.0 * a * (A @ meanY) + a * a
    s2 = jnp.sqrt(Ez2 - m2 * m2 + _EPS)
    A2 = (g2 / s2)[:, None] * A
    a2 = g2 * (a - m2) / s2 + be2

    A3 = Wc0 @ A2
    a3 = Wc0 @ a2 + bc0
    m3 = A3 @ meanY + a3
    Ez3 = _qdiag(A3, MY) + 2.0 * a3 * (A3 @ meanY) + a3 * a3
    s3 = jnp.sqrt(Ez3 - m3 * m3 + _EPS)
    A3f = (gc0 / s3)[:, None] * A3
    a3f = (gc0 * (a3 - m3) / s3 + bec0)[:, None]
    A3p = A3f[:, :64]
    A3u = A3f[:, 64:]

    m3aug = pl.pallas_call(
        _p2_kernel,
        grid=(B, NB),
        in_specs=[
            pl.BlockSpec((None, Cin, T), lambda b, n: (b, 0, n)),
            pl.BlockSpec((None, 3, T), lambda b, n: (b, 0, n)),
            pl.BlockSpec((8, 3), lambda b, n: (0, 0)),
            pl.BlockSpec((8, 1), lambda b, n: (0, 0)),
            pl.BlockSpec((64, 64), lambda b, n: (0, 0)),
            pl.BlockSpec((64, 8), lambda b, n: (0, 0)),
            pl.BlockSpec((64, 1), lambda b, n: (0, 0)),
        ],
        out_specs=pl.BlockSpec((65, 65), lambda b, n: (0, 0)),
        out_shape=jax.ShapeDtypeStruct((65, 65), jnp.float32),
        compiler_params=cp,
    )(points, xyz, Au, cu, A3p, A3u, a3f)

    mean3 = m3aug[64, :64] / cnt
    M3 = m3aug[:64, :64] / cnt
    m4 = Wc1 @ mean3 + bc1
    Ez4 = _qdiag(Wc1, M3) + 2.0 * bc1 * (Wc1 @ mean3) + bc1 * bc1
    s4 = jnp.sqrt(Ez4 - m4 * m4 + _EPS)
    A4 = (gc1 / s4)[:, None] * Wc1
    a4f = (gc1 * (bc1 - m4) / s4 + bec1)[:, None]

    m4aug = pl.pallas_call(
        _p3_kernel,
        grid=(B, NB),
        in_specs=[
            pl.BlockSpec((None, Cin, T), lambda b, n: (b, 0, n)),
            pl.BlockSpec((None, 3, T), lambda b, n: (b, 0, n)),
            pl.BlockSpec((8, 3), lambda b, n: (0, 0)),
            pl.BlockSpec((8, 1), lambda b, n: (0, 0)),
            pl.BlockSpec((64, 64), lambda b, n: (0, 0)),
            pl.BlockSpec((64, 8), lambda b, n: (0, 0)),
            pl.BlockSpec((64, 1), lambda b, n: (0, 0)),
            pl.BlockSpec((64, 64), lambda b, n: (0, 0)),
            pl.BlockSpec((64, 1), lambda b, n: (0, 0)),
        ],
        out_specs=pl.BlockSpec((None, 65, 65), lambda b, n: (b, 0, 0)),
        out_shape=jax.ShapeDtypeStruct((B, 65, 65), jnp.float32),
        compiler_params=cp,
    )(points, xyz, Au, cu, A3p, A3u, a3f, A4, a4f)

    y_b = m4aug[:, 64, :64] / float(N)
    M4 = m4aug[:, :64, :64] / float(N)

    yp = jnp.pad(y_b, ((0, 0), (1, 1)))
    yc = wk[0] * yp[:, :-2] + wk[1] * yp[:, 1:-1] + wk[2] * yp[:, 2:]
    sig = jax.nn.sigmoid(yc)

    Web = We[None, :, :] * sig[:, None, :]
    mE_b = jnp.einsum('boc,bc->bo', Web, y_b) + bE[None, :]
    mE = jnp.mean(mE_b, axis=0)
    Ez5 = jnp.mean(
        jnp.einsum('boc,bcd,bod->bo', Web, M4, Web)
        + 2.0 * bE[None, :] * (mE_b - bE[None, :]) + (bE * bE)[None, :],
        axis=0)
    sE = jnp.sqrt(Ez5 - mE * mE + _EPS)
    Wfb = (gE / sE)[None, :, None] * Web
    shift = gE * (bE - mE) / sE + beE

    rawmax = pl.pallas_call(
        _p4_kernel,
        grid=(B, NB),
        in_specs=[
            pl.BlockSpec((None, Cin, T), lambda b, n: (b, 0, n)),
            pl.BlockSpec((None, 3, T), lambda b, n: (b, 0, n)),
            pl.BlockSpec((8, 3), lambda b, n: (0, 0)),
            pl.BlockSpec((8, 1), lambda b, n: (0, 0)),
            pl.BlockSpec((64, 64), lambda b, n: (0, 0)),
            pl.BlockSpec((64, 8), lambda b, n: (0, 0)),
            pl.BlockSpec((64, 1), lambda b, n: (0, 0)),
            pl.BlockSpec((64, 64), lambda b, n: (0, 0)),
            pl.BlockSpec((64, 1), lambda b, n: (0, 0)),
            pl.BlockSpec((None, 128, 64), lambda b, n: (b, 0, 0)),
        ],
        out_specs=pl.BlockSpec((None, 128, 1), lambda b, n: (b, 0, 0)),
        out_shape=jax.ShapeDtypeStruct((B, 128, 1), jnp.float32),
        compiler_params=cp,
    )(points, xyz, Au, cu, A3p, A3u, a3f, A4, a4f, Wfb)

    new_features = rawmax + shift[None, :, None]
    new_xyz = jnp.zeros((B, 3, 1), dtype=xyz.dtype)
    return new_xyz, new_features


# speedup vs baseline: 2.8019x; 1.2046x over previous
import jax
import jax.numpy as jnp
from jax.experimental import pallas as pl
from jax.experimental.pallas import tpu as pltpu

_EPS = 1e-5
_T = 16384


def _first(b, n):
    return (b == 0) & (n == 0)


def _mm(a, b):
    return jax.lax.dot_general(
        a.astype(jnp.bfloat16), b.astype(jnp.bfloat16),
        (((1,), (0,)), ((), ())), preferred_element_type=jnp.float32)


def _outer(a):
    ab = a.astype(jnp.bfloat16)
    return jax.lax.dot_general(ab, ab, (((1,), (1,)), ((), ())),
                               preferred_element_type=jnp.float32)


def _p0_kernel(x_ref, mom_ref, sum_ref):
    x = x_ref[...]
    m = jax.lax.dot_general(x, x, (((1,), (1,)), ((), ())),
                            preferred_element_type=jnp.float32)
    s = jnp.sum(x, axis=1, keepdims=True)
    f = _first(pl.program_id(0), pl.program_id(1))

    @pl.when(f)
    def _():
        mom_ref[...] = m
        sum_ref[...] = s

    @pl.when(jnp.logical_not(f))
    def _():
        mom_ref[...] = mom_ref[...] + m
        sum_ref[...] = sum_ref[...] + s


def _u_of(x_ref, au_ref, cu_ref):
    return jnp.maximum(_mm(au_ref[...], x_ref[...]) + cu_ref[...], 0.0)


def _p1_kernel(p_ref, x_ref, au_ref, cu_ref, out_ref):
    u = _u_of(x_ref, au_ref, cu_ref)
    p = p_ref[...]
    ones = jnp.ones((1, p.shape[1]), jnp.float32)
    ya = jnp.concatenate([p, u, ones], axis=0)
    m = _outer(ya)
    f = _first(pl.program_id(0), pl.program_id(1))

    @pl.when(f)
    def _():
        out_ref[...] = m

    @pl.when(jnp.logical_not(f))
    def _():
        out_ref[...] = out_ref[...] + m


def _x3_of(p_ref, x_ref, au_ref, cu_ref, a3p_ref, a3u_ref, a3c_ref):
    u = _u_of(x_ref, au_ref, cu_ref)
    h = _mm(a3p_ref[...], p_ref[...]) + _mm(a3u_ref[...], u) + a3c_ref[...]
    return jnp.maximum(h, 0.0)


def _p2_kernel(p_ref, x_ref, au_ref, cu_ref, a3p_ref, a3u_ref, a3c_ref,
               out_ref):
    x3 = _x3_of(p_ref, x_ref, au_ref, cu_ref, a3p_ref, a3u_ref, a3c_ref)
    ones = jnp.ones((1, x3.shape[1]), jnp.float32)
    xa = jnp.concatenate([x3, ones], axis=0)
    m = _outer(xa)
    f = _first(pl.program_id(0), pl.program_id(1))

    @pl.when(f)
    def _():
        out_ref[...] = m

    @pl.when(jnp.logical_not(f))
    def _():
        out_ref[...] = out_ref[...] + m


def _p3_kernel(p_ref, x_ref, au_ref, cu_ref, a3p_ref, a3u_ref, a3c_ref,
               a4_ref, a4c_ref, out_ref):
    x3 = _x3_of(p_ref, x_ref, au_ref, cu_ref, a3p_ref, a3u_ref, a3c_ref)
    x4 = jnp.maximum(_mm(a4_ref[...], x3) + a4c_ref[...], 0.0)
    ones = jnp.ones((1, x4.shape[1]), jnp.float32)
    xa = jnp.concatenate([x4, ones], axis=0)
    m = _outer(xa)
    f = pl.program_id(1) == 0

    @pl.when(f)
    def _():
        out_ref[...] = m

    @pl.when(jnp.logical_not(f))
    def _():
        out_ref[...] = out_ref[...] + m


def _p4_kernel(p_ref, x_ref, au_ref, cu_ref, a3p_ref, a3u_ref, a3c_ref,
               a4_ref, a4c_ref, wf_ref, out_ref):
    x3 = _x3_of(p_ref, x_ref, au_ref, cu_ref, a3p_ref, a3u_ref, a3c_ref)
    x4 = jnp.maximum(_mm(a4_ref[...], x3) + a4c_ref[...], 0.0)
    z5 = _mm(wf_ref[...], x4)
    pm = jnp.max(z5, axis=1, keepdims=True)
    f = pl.program_id(1) == 0

    @pl.when(f)
    def _():
        out_ref[...] = pm

    @pl.when(jnp.logical_not(f))
    def _():
        out_ref[...] = jnp.maximum(out_ref[...], pm)


def _qdiag(A, M):
    return jnp.sum((A @ M) * A, axis=1)


def kernel(xyz, points, W0, b0, g0, be0, W1, b1, g1, be1, W2, b2, g2, be2,
           Wc0, bc0, gc0, bec0, Wc1, bc1, gc1, bec1, wk, We, bE, gE, beE):
    B, _, N = xyz.shape
    Cin = points.shape[1]
    T = _T
    NB = N // T
    cnt = float(B * N)
    cp = pltpu.CompilerParams(dimension_semantics=("arbitrary", "arbitrary"))

    xyz = xyz.astype(jnp.float32)
    points = points.astype(jnp.float32)

    mom_x, sum_x = pl.pallas_call(
        _p0_kernel,
        grid=(B, NB),
        in_specs=[pl.BlockSpec((None, 3, T), lambda b, n: (b, 0, n))],
        out_specs=[pl.BlockSpec((3, 3), lambda b, n: (0, 0)),
                   pl.BlockSpec((3, 1), lambda b, n: (0, 0))],
        out_shape=[jax.ShapeDtypeStruct((3, 3), jnp.float32),
                   jax.ShapeDtypeStruct((3, 1), jnp.float32)],
        compiler_params=cp,
    )(xyz)

    mean_x = sum_x[:, 0] / cnt
    Mx = mom_x / cnt
    m0 = W0 @ mean_x + b0
    Ez0 = _qdiag(W0, Mx) + 2.0 * b0 * (W0 @ mean_x) + b0 * b0
    s0 = jnp.sqrt(Ez0 - m0 * m0 + _EPS)
    Au = (g0 / s0)[:, None] * W0
    cu = (g0 * (b0 - m0) / s0 + be0)[:, None]

    maug = pl.pallas_call(
        _p1_kernel,
        grid=(B, NB),
        in_specs=[
            pl.BlockSpec((None, Cin, T), lambda b, n: (b, 0, n)),
            pl.BlockSpec((None, 3, T), lambda b, n: (b, 0, n)),
            pl.BlockSpec((8, 3), lambda b, n: (0, 0)),
            pl.BlockSpec((8, 1), lambda b, n: (0, 0)),
        ],
        out_specs=pl.BlockSpec((73, 73), lambda b, n: (0, 0)),
        out_shape=jax.ShapeDtypeStruct((73, 73), jnp.float32),
        compiler_params=cp,
    )(points, xyz, Au, cu)

    MY = maug[:72, :72] / cnt
    meanY = maug[72, :72] / cnt

    mean_u = meanY[64:]
    Muu = MY[64:, 64:]
    m1 = W1 @ mean_u + b1
    Ez1 = _qdiag(W1, Muu) + 2.0 * b1 * (W1 @ mean_u) + b1 * b1
    s1 = jnp.sqrt(Ez1 - m1 * m1 + _EPS)
    W1f = (g1 / s1)[:, None] * W1
    c1f = g1 * (b1 - m1) / s1 + be1

    A = jnp.concatenate([W2, W2 @ W1f], axis=1)
    a = W2 @ c1f + b2
    m2 = A @ meanY + a
    Ez2 = _qdiag(A, MY) + 2.0 * a * (A @ meanY) + a * a
    s2 = jnp.sqrt(Ez2 - m2 * m2 + _EPS)
    A2 = (g2 / s2)[:, None] * A
    a2 = g2 * (a - m2) / s2 + be2

    A3 = Wc0 @ A2
    a3 = Wc0 @ a2 + bc0
    m3 = A3 @ meanY + a3
    Ez3 = _qdiag(A3, MY) + 2.0 * a3 * (A3 @ meanY) + a3 * a3
    s3 = jnp.sqrt(Ez3 - m3 * m3 + _EPS)
    A3f = (gc0 / s3)[:, None] * A3
    a3f = (gc0 * (a3 - m3) / s3 + bec0)[:, None]
    A3p = A3f[:, :64]
    A3u = A3f[:, 64:]

    m3aug = pl.pallas_call(
        _p2_kernel,
        grid=(B, NB),
        in_specs=[
            pl.BlockSpec((None, Cin, T), lambda b, n: (b, 0, n)),
            pl.BlockSpec((None, 3, T), lambda b, n: (b, 0, n)),
            pl.BlockSpec((8, 3), lambda b, n: (0, 0)),
            pl.BlockSpec((8, 1), lambda b, n: (0, 0)),
            pl.BlockSpec((64, 64), lambda b, n: (0, 0)),
            pl.BlockSpec((64, 8), lambda b, n: (0, 0)),
            pl.BlockSpec((64, 1), lambda b, n: (0, 0)),
        ],
        out_specs=pl.BlockSpec((65, 65), lambda b, n: (0, 0)),
        out_shape=jax.ShapeDtypeStruct((65, 65), jnp.float32),
        compiler_params=cp,
    )(points, xyz, Au, cu, A3p, A3u, a3f)

    mean3 = m3aug[64, :64] / cnt
    M3 = m3aug[:64, :64] / cnt
    m4 = Wc1 @ mean3 + bc1
    Ez4 = _qdiag(Wc1, M3) + 2.0 * bc1 * (Wc1 @ mean3) + bc1 * bc1
    s4 = jnp.sqrt(Ez4 - m4 * m4 + _EPS)
    A4 = (gc1 / s4)[:, None] * Wc1
    a4f = (gc1 * (bc1 - m4) / s4 + bec1)[:, None]

    m4aug = pl.pallas_call(
        _p3_kernel,
        grid=(B, NB),
        in_specs=[
            pl.BlockSpec((None, Cin, T), lambda b, n: (b, 0, n)),
            pl.BlockSpec((None, 3, T), lambda b, n: (b, 0, n)),
            pl.BlockSpec((8, 3), lambda b, n: (0, 0)),
            pl.BlockSpec((8, 1), lambda b, n: (0, 0)),
            pl.BlockSpec((64, 64), lambda b, n: (0, 0)),
            pl.BlockSpec((64, 8), lambda b, n: (0, 0)),
            pl.BlockSpec((64, 1), lambda b, n: (0, 0)),
            pl.BlockSpec((64, 64), lambda b, n: (0, 0)),
            pl.BlockSpec((64, 1), lambda b, n: (0, 0)),
        ],
        out_specs=pl.BlockSpec((None, 65, 65), lambda b, n: (b, 0, 0)),
        out_shape=jax.ShapeDtypeStruct((B, 65, 65), jnp.float32),
        compiler_params=cp,
    )(points, xyz, Au, cu, A3p, A3u, a3f, A4, a4f)

    y_b = m4aug[:, 64, :64] / float(N)
    M4 = m4aug[:, :64, :64] / float(N)

    yp = jnp.pad(y_b, ((0, 0), (1, 1)))
    yc = wk[0] * yp[:, :-2] + wk[1] * yp[:, 1:-1] + wk[2] * yp[:, 2:]
    sig = jax.nn.sigmoid(yc)

    Web = We[None, :, :] * sig[:, None, :]
    mE_b = jnp.einsum('boc,bc->bo', Web, y_b) + bE[None, :]
    mE = jnp.mean(mE_b, axis=0)
    Ez5 = jnp.mean(
        jnp.einsum('boc,bcd,bod->bo', Web, M4, Web)
        + 2.0 * bE[None, :] * (mE_b - bE[None, :]) + (bE * bE)[None, :],
        axis=0)
    sE = jnp.sqrt(Ez5 - mE * mE + _EPS)
    Wfb = (gE / sE)[None, :, None] * Web
    shift = gE * (bE - mE) / sE + beE

    rawmax = pl.pallas_call(
        _p4_kernel,
        grid=(B, NB),
        in_specs=[
            pl.BlockSpec((None, Cin, T), lambda b, n: (b, 0, n)),
            pl.BlockSpec((None, 3, T), lambda b, n: (b, 0, n)),
            pl.BlockSpec((8, 3), lambda b, n: (0, 0)),
            pl.BlockSpec((8, 1), lambda b, n: (0, 0)),
            pl.BlockSpec((64, 64), lambda b, n: (0, 0)),
            pl.BlockSpec((64, 8), lambda b, n: (0, 0)),
            pl.BlockSpec((64, 1), lambda b, n: (0, 0)),
            pl.BlockSpec((64, 64), lambda b, n: (0, 0)),
            pl.BlockSpec((64, 1), lambda b, n: (0, 0)),
            pl.BlockSpec((None, 128, 64), lambda b, n: (b, 0, 0)),
        ],
        out_specs=pl.BlockSpec((None, 128, 1), lambda b, n: (b, 0, 0)),
        out_shape=jax.ShapeDtypeStruct((B, 128, 1), jnp.float32),
        compiler_params=cp,
    )(points, xyz, Au, cu, A3p, A3u, a3f, A4, a4f, Wfb)

    new_features = rawmax + shift[None, :, None]
    new_xyz = jnp.zeros((B, 3, 1), dtype=xyz.dtype)
    return new_xyz, new_features


# speedup vs baseline: 2.9889x; 1.0667x over previous
import jax
import jax.numpy as jnp
from jax.experimental import pallas as pl
from jax.experimental.pallas import tpu as pltpu

_EPS = 1e-5
_T = 32768


def _first(b, n):
    return (b == 0) & (n == 0)


def _mm(a, b):
    return jax.lax.dot_general(
        a.astype(jnp.bfloat16), b.astype(jnp.bfloat16),
        (((1,), (0,)), ((), ())), preferred_element_type=jnp.float32)


def _outer(a):
    ab = a.astype(jnp.bfloat16)
    return jax.lax.dot_general(ab, ab, (((1,), (1,)), ((), ())),
                               preferred_element_type=jnp.float32)


def _p0_kernel(x_ref, mom_ref, sum_ref):
    x = x_ref[...]
    m = jax.lax.dot_general(x, x, (((1,), (1,)), ((), ())),
                            preferred_element_type=jnp.float32)
    s = jnp.sum(x, axis=1, keepdims=True)
    f = _first(pl.program_id(0), pl.program_id(1))

    @pl.when(f)
    def _():
        mom_ref[...] = m
        sum_ref[...] = s

    @pl.when(jnp.logical_not(f))
    def _():
        mom_ref[...] = mom_ref[...] + m
        sum_ref[...] = sum_ref[...] + s


def _u_of(x_ref, au_ref, cu_ref):
    return jnp.maximum(_mm(au_ref[...], x_ref[...]) + cu_ref[...], 0.0)


def _p1_kernel(p_ref, x_ref, au_ref, cu_ref, out_ref):
    u = _u_of(x_ref, au_ref, cu_ref)
    p = p_ref[...]
    ones = jnp.ones((1, p.shape[1]), jnp.float32)
    ya = jnp.concatenate([p, u, ones], axis=0)
    m = _outer(ya)
    f = _first(pl.program_id(0), pl.program_id(1))

    @pl.when(f)
    def _():
        out_ref[...] = m

    @pl.when(jnp.logical_not(f))
    def _():
        out_ref[...] = out_ref[...] + m


def _x3_of(p_ref, x_ref, au_ref, cu_ref, a3p_ref, a3u_ref, a3c_ref):
    u = _u_of(x_ref, au_ref, cu_ref)
    h = _mm(a3p_ref[...], p_ref[...]) + _mm(a3u_ref[...], u) + a3c_ref[...]
    return jnp.maximum(h, 0.0)


def _p2_kernel(p_ref, x_ref, au_ref, cu_ref, a3p_ref, a3u_ref, a3c_ref,
               out_ref):
    x3 = _x3_of(p_ref, x_ref, au_ref, cu_ref, a3p_ref, a3u_ref, a3c_ref)
    ones = jnp.ones((1, x3.shape[1]), jnp.float32)
    xa = jnp.concatenate([x3, ones], axis=0)
    m = _outer(xa)
    f = _first(pl.program_id(0), pl.program_id(1))

    @pl.when(f)
    def _():
        out_ref[...] = m

    @pl.when(jnp.logical_not(f))
    def _():
        out_ref[...] = out_ref[...] + m


def _p3_kernel(p_ref, x_ref, au_ref, cu_ref, a3p_ref, a3u_ref, a3c_ref,
               a4_ref, a4c_ref, out_ref):
    x3 = _x3_of(p_ref, x_ref, au_ref, cu_ref, a3p_ref, a3u_ref, a3c_ref)
    x4 = jnp.maximum(_mm(a4_ref[...], x3) + a4c_ref[...], 0.0)
    ones = jnp.ones((1, x4.shape[1]), jnp.float32)
    xa = jnp.concatenate([x4, ones], axis=0)
    m = _outer(xa)
    f = pl.program_id(1) == 0

    @pl.when(f)
    def _():
        out_ref[...] = m

    @pl.when(jnp.logical_not(f))
    def _():
        out_ref[...] = out_ref[...] + m


def _p4_kernel(p_ref, x_ref, au_ref, cu_ref, a3p_ref, a3u_ref, a3c_ref,
               a4_ref, a4c_ref, wf_ref, out_ref):
    x3 = _x3_of(p_ref, x_ref, au_ref, cu_ref, a3p_ref, a3u_ref, a3c_ref)
    x4 = jnp.maximum(_mm(a4_ref[...], x3) + a4c_ref[...], 0.0)
    z5 = _mm(wf_ref[...], x4)
    pm = jnp.max(z5, axis=1, keepdims=True)
    f = pl.program_id(1) == 0

    @pl.when(f)
    def _():
        out_ref[...] = pm

    @pl.when(jnp.logical_not(f))
    def _():
        out_ref[...] = jnp.maximum(out_ref[...], pm)


def _qdiag(A, M):
    return jnp.sum((A @ M) * A, axis=1)


def kernel(xyz, points, W0, b0, g0, be0, W1, b1, g1, be1, W2, b2, g2, be2,
           Wc0, bc0, gc0, bec0, Wc1, bc1, gc1, bec1, wk, We, bE, gE, beE):
    B, _, N = xyz.shape
    Cin = points.shape[1]
    T = _T
    NB = N // T
    cnt = float(B * N)
    cp = pltpu.CompilerParams(dimension_semantics=("arbitrary", "arbitrary"))

    xyz = xyz.astype(jnp.float32)
    points = points.astype(jnp.float32)

    mom_x, sum_x = pl.pallas_call(
        _p0_kernel,
        grid=(B, NB),
        in_specs=[pl.BlockSpec((None, 3, T), lambda b, n: (b, 0, n))],
        out_specs=[pl.BlockSpec((3, 3), lambda b, n: (0, 0)),
                   pl.BlockSpec((3, 1), lambda b, n: (0, 0))],
        out_shape=[jax.ShapeDtypeStruct((3, 3), jnp.float32),
                   jax.ShapeDtypeStruct((3, 1), jnp.float32)],
        compiler_params=cp,
    )(xyz)

    mean_x = sum_x[:, 0] / cnt
    Mx = mom_x / cnt
    m0 = W0 @ mean_x + b0
    Ez0 = _qdiag(W0, Mx) + 2.0 * b0 * (W0 @ mean_x) + b0 * b0
    s0 = jnp.sqrt(Ez0 - m0 * m0 + _EPS)
    Au = (g0 / s0)[:, None] * W0
    cu = (g0 * (b0 - m0) / s0 + be0)[:, None]

    maug = pl.pallas_call(
        _p1_kernel,
        grid=(B, NB),
        in_specs=[
            pl.BlockSpec((None, Cin, T), lambda b, n: (b, 0, n)),
            pl.BlockSpec((None, 3, T), lambda b, n: (b, 0, n)),
            pl.BlockSpec((8, 3), lambda b, n: (0, 0)),
            pl.BlockSpec((8, 1), lambda b, n: (0, 0)),
        ],
        out_specs=pl.BlockSpec((73, 73), lambda b, n: (0, 0)),
        out_shape=jax.ShapeDtypeStruct((73, 73), jnp.float32),
        compiler_params=cp,
    )(points, xyz, Au, cu)

    MY = maug[:72, :72] / cnt
    meanY = maug[72, :72] / cnt

    mean_u = meanY[64:]
    Muu = MY[64:, 64:]
    m1 = W1 @ mean_u + b1
    Ez1 = _qdiag(W1, Muu) + 2.0 * b1 * (W1 @ mean_u) + b1 * b1
    s1 = jnp.sqrt(Ez1 - m1 * m1 + _EPS)
    W1f = (g1 / s1)[:, None] * W1
    c1f = g1 * (b1 - m1) / s1 + be1

    A = jnp.concatenate([W2, W2 @ W1f], axis=1)
    a = W2 @ c1f + b2
    m2 = A @ meanY + a
    Ez2 = _qdiag(A, MY) + 2.0 * a * (A @ meanY) + a * a
    s2 = jnp.sqrt(Ez2 - m2 * m2 + _EPS)
    A2 = (g2 / s2)[:, None] * A
    a2 = g2 * (a - m2) / s2 + be2

    A3 = Wc0 @ A2
    a3 = Wc0 @ a2 + bc0
    m3 = A3 @ meanY + a3
    Ez3 = _qdiag(A3, MY) + 2.0 * a3 * (A3 @ meanY) + a3 * a3
    s3 = jnp.sqrt(Ez3 - m3 * m3 + _EPS)
    A3f = (gc0 / s3)[:, None] * A3
    a3f = (gc0 * (a3 - m3) / s3 + bec0)[:, None]
    A3p = A3f[:, :64]
    A3u = A3f[:, 64:]

    m3aug = pl.pallas_call(
        _p2_kernel,
        grid=(B, NB),
        in_specs=[
            pl.BlockSpec((None, Cin, T), lambda b, n: (b, 0, n)),
            pl.BlockSpec((None, 3, T), lambda b, n: (b, 0, n)),
            pl.BlockSpec((8, 3), lambda b, n: (0, 0)),
            pl.BlockSpec((8, 1), lambda b, n: (0, 0)),
            pl.BlockSpec((64, 64), lambda b, n: (0, 0)),
            pl.BlockSpec((64, 8), lambda b, n: (0, 0)),
            pl.BlockSpec((64, 1), lambda b, n: (0, 0)),
        ],
        out_specs=pl.BlockSpec((65, 65), lambda b, n: (0, 0)),
        out_shape=jax.ShapeDtypeStruct((65, 65), jnp.float32),
        compiler_params=cp,
    )(points, xyz, Au, cu, A3p, A3u, a3f)

    mean3 = m3aug[64, :64] / cnt
    M3 = m3aug[:64, :64] / cnt
    m4 = Wc1 @ mean3 + bc1
    Ez4 = _qdiag(Wc1, M3) + 2.0 * bc1 * (Wc1 @ mean3) + bc1 * bc1
    s4 = jnp.sqrt(Ez4 - m4 * m4 + _EPS)
    A4 = (gc1 / s4)[:, None] * Wc1
    a4f = (gc1 * (bc1 - m4) / s4 + bec1)[:, None]

    m4aug = pl.pallas_call(
        _p3_kernel,
        grid=(B, NB),
        in_specs=[
            pl.BlockSpec((None, Cin, T), lambda b, n: (b, 0, n)),
            pl.BlockSpec((None, 3, T), lambda b, n: (b, 0, n)),
            pl.BlockSpec((8, 3), lambda b, n: (0, 0)),
            pl.BlockSpec((8, 1), lambda b, n: (0, 0)),
            pl.BlockSpec((64, 64), lambda b, n: (0, 0)),
            pl.BlockSpec((64, 8), lambda b, n: (0, 0)),
            pl.BlockSpec((64, 1), lambda b, n: (0, 0)),
            pl.BlockSpec((64, 64), lambda b, n: (0, 0)),
            pl.BlockSpec((64, 1), lambda b, n: (0, 0)),
        ],
        out_specs=pl.BlockSpec((None, 65, 65), lambda b, n: (b, 0, 0)),
        out_shape=jax.ShapeDtypeStruct((B, 65, 65), jnp.float32),
        compiler_params=cp,
    )(points, xyz, Au, cu, A3p, A3u, a3f, A4, a4f)

    y_b = m4aug[:, 64, :64] / float(N)
    M4 = m4aug[:, :64, :64] / float(N)

    yp = jnp.pad(y_b, ((0, 0), (1, 1)))
    yc = wk[0] * yp[:, :-2] + wk[1] * yp[:, 1:-1] + wk[2] * yp[:, 2:]
    sig = jax.nn.sigmoid(yc)

    Web = We[None, :, :] * sig[:, None, :]
    mE_b = jnp.einsum('boc,bc->bo', Web, y_b) + bE[None, :]
    mE = jnp.mean(mE_b, axis=0)
    Ez5 = jnp.mean(
        jnp.einsum('boc,bcd,bod->bo', Web, M4, Web)
        + 2.0 * bE[None, :] * (mE_b - bE[None, :]) + (bE * bE)[None, :],
        axis=0)
    sE = jnp.sqrt(Ez5 - mE * mE + _EPS)
    Wfb = (gE / sE)[None, :, None] * Web
    shift = gE * (bE - mE) / sE + beE

    rawmax = pl.pallas_call(
        _p4_kernel,
        grid=(B, NB),
        in_specs=[
            pl.BlockSpec((None, Cin, T), lambda b, n: (b, 0, n)),
            pl.BlockSpec((None, 3, T), lambda b, n: (b, 0, n)),
            pl.BlockSpec((8, 3), lambda b, n: (0, 0)),
            pl.BlockSpec((8, 1), lambda b, n: (0, 0)),
            pl.BlockSpec((64, 64), lambda b, n: (0, 0)),
            pl.BlockSpec((64, 8), lambda b, n: (0, 0)),
            pl.BlockSpec((64, 1), lambda b, n: (0, 0)),
            pl.BlockSpec((64, 64), lambda b, n: (0, 0)),
            pl.BlockSpec((64, 1), lambda b, n: (0, 0)),
            pl.BlockSpec((None, 128, 64), lambda b, n: (b, 0, 0)),
        ],
        out_specs=pl.BlockSpec((None, 128, 1), lambda b, n: (b, 0, 0)),
        out_shape=jax.ShapeDtypeStruct((B, 128, 1), jnp.float32),
        compiler_params=cp,
    )(points, xyz, Au, cu, A3p, A3u, a3f, A4, a4f, Wfb)

    new_features = rawmax + shift[None, :, None]
    new_xyz = jnp.zeros((B, 3, 1), dtype=xyz.dtype)
    return new_xyz, new_features
